# Initial kernel scaffold; baseline (speedup 1.0000x reference)
import jax
import jax.numpy as jnp
from jax.experimental import pallas as pl


def kernel(xyz, dense, hash_table):
    raise NotImplementedError("write your pallas kernel here")


# speedup vs baseline: 9.2285x; 9.2285x over previous
import functools

import numpy as np
import jax
import jax.numpy as jnp
from jax import lax
from jax.experimental import pallas as pl
from jax.experimental.pallas import tpu as pltpu
from jax.experimental.pallas import tpu_sc as plsc

_NL = 16
_SH = 6
_T = 524309
_RES = [16, 22, 30, 42, 58, 80, 110, 152, 210, 290, 400, 553, 763, 1053, 1453, 2005]
_DBASE = [0, 4096, 14744, 41744, 115832, 310944]
_P1, _P2 = 19349663, 83492791
_P1H, _P1L = _P1 >> 16, _P1 & 0xFFFF
_P2H, _P2L = _P2 >> 16, _P2 & 0xFFFF
_INVT = np.float32(1.0) / np.float32(_T)
_MIN32 = np.int32(-(2 ** 31))
_SCALE = [float(np.float32(1.0) / np.float32(1.0 / (r - 1.0))) for r in _RES]

_C = 16
_NW = 32
_PPT = 4096
_NCHUNK = _PPT // _C
_GROW = 8 * _C
_OD = 3 + 2 * _NL


def _i32(v):
    return np.int32(v)


def _f32(v):
    return np.float32(v)


def _prod16(v, ph, plo):
    m = v * _i32(ph)
    lp = v * _i32(plo)
    a = m << 16
    lo = a + lp
    carry = jnp.where((lo ^ _MIN32) < (a ^ _MIN32), _i32(1), _i32(0))
    hi = (m >> 16) + carry
    return hi, lo


def _mod_t(hi, lo):
    lof = lo.astype(jnp.float32) + jnp.where(lo < 0, _f32(2.0 ** 32), _f32(0.0))
    vf = hi.astype(jnp.float32) * _f32(2.0 ** 32) + lof
    q = (vf * _INVT).astype(jnp.int32)
    r = lo - q * _i32(_T)
    r = r + jnp.where(r < 0, _i32(_T), _i32(0))
    r = r - jnp.where(r >= _T, _i32(_T), _i32(0))
    return r


def _body(x_hbm, y_hbm, z_hbm, dense_hbm, hash_hbm, out_hbm,
          xyz_v, idx_v, val_v, out_v, gsem):
    wid = lax.axis_index("s") * _i32(2) + lax.axis_index("c")
    tile_base = wid * _i32(_PPT)

    iota = lax.iota(jnp.int32, 16)
    czero = jnp.zeros((16,), jnp.int32)
    cone = jnp.full((16,), 1, jnp.int32)

    for d, src in enumerate((x_hbm, y_hbm, z_hbm)):
        pltpu.sync_copy(src.at[pl.ds(tile_base, _PPT)],
                        xyz_v.at[pl.ds(d * _PPT, _PPT)])

    def _norm(j16):
        x = xyz_v[pl.ds(j16, 16)] * _f32(0.5) + _f32(0.5)
        y = xyz_v[pl.ds(j16 + _i32(_PPT), 16)] * _f32(0.5) + _f32(0.5)
        z = xyz_v[pl.ds(j16 + _i32(2 * _PPT), 16)] * _f32(0.5) + _f32(0.5)
        return x, y, z

    def _it(i):
        slot = i & _i32(1)
        pslot = _i32(1) - slot
        in_act = i < _i32(_NCHUNK)
        out_act = i >= _i32(1)

        @pl.when(in_act)
        def _phase_a():
            x, y, z = _norm(i * _i32(_C))
            for l in range(_NL):
                r = _RES[l]
                fx = x * _f32(_SCALE[l])
                fy = y * _f32(_SCALE[l])
                fz = z * _f32(_SCALE[l])
                ix0 = fx.astype(jnp.int32)
                iy0 = fy.astype(jnp.int32)
                iz0 = fz.astype(jnp.int32)
                ix1 = jnp.minimum((fx + _f32(1.0)).astype(jnp.int32), _i32(r - 1))
                iy1 = jnp.minimum((fy + _f32(1.0)).astype(jnp.int32), _i32(r - 1))
                iz1 = jnp.minimum((fz + _f32(1.0)).astype(jnp.int32), _i32(r - 1))
                row = slot * _i32(_NL) + _i32(l)
                if l < _SH:
                    for k in range(8):
                        xk = ix1 if (k >> 2) & 1 else ix0
                        yk = iy1 if (k >> 1) & 1 else iy0
                        zk = iz1 if k & 1 else iz0
                        idx = xk * _i32(r * r) + yk * _i32(r) + zk + _i32(_DBASE[l])
                        idx_v[row, pl.ds(k * 16, 16)] = idx
                else:
                    yp = (_prod16(iy0, _P1H, _P1L), _prod16(iy1, _P1H, _P1L))
                    zp = (_prod16(iz0, _P2H, _P2L), _prod16(iz1, _P2H, _P2L))
                    xv = (ix0 ^ _i32(1), ix1 ^ _i32(1))
                    lbase = _i32((l - _SH) * _T)
                    for dy in range(2):
                        for dz in range(2):
                            lo_b = yp[dy][1] ^ zp[dz][1]
                            hi_b = yp[dy][0] ^ zp[dz][0]
                            rb = _mod_t(hi_b, lo_b)
                            b11 = lo_b & _i32(0x7FF)
                            for dx in range(2):
                                rr = rb + ((b11 ^ xv[dx]) - b11)
                                rr = rr + jnp.where(rr < 0, _i32(_T), _i32(0))
                                rr = rr - jnp.where(rr >= _T, _i32(_T), _i32(0))
                                k = (dx << 2) | (dy << 1) | dz
                                idx_v[row, pl.ds(k * 16, 16)] = rr + lbase

            for l in range(_NL):
                tbl = dense_hbm if l < _SH else hash_hbm
                pltpu.sync_copy(
                    tbl.at[idx_v.at[slot * _i32(_NL) + _i32(l)]],
                    val_v.at[pl.ds((slot * _i32(_NL) + _i32(l)) * _i32(_GROW),
                                   _GROW)])

        @pl.when(out_act)
        def _phase_c():
            j = i - _i32(1)
            x, y, z = _norm(j * _i32(_C))
            rows = pslot * _i32(_C) + iota
            plsc.store_scatter(out_v, [rows, czero], x)
            plsc.store_scatter(out_v, [rows, cone], y)
            plsc.store_scatter(out_v, [rows, jnp.full((16,), 2, jnp.int32)], z)
            for l in range(_NL):
                fx = x * _f32(_SCALE[l])
                fy = y * _f32(_SCALE[l])
                wx = fx - fx.astype(jnp.int32).astype(jnp.float32)
                wy = fy - fy.astype(jnp.int32).astype(jnp.float32)
                ux = _f32(1.0) - wx
                uy = _f32(1.0) - wy
                wj = (ux * uy, ux * wy, wx * uy, wx * wy)
                base = (pslot * _i32(_NL) + _i32(l)) * _i32(_GROW) + iota
                acc0 = jnp.zeros((16,), jnp.float32)
                acc1 = jnp.zeros((16,), jnp.float32)
                for jc in range(4):
                    r0 = base + _i32((2 * jc) * 16)
                    r1 = base + _i32((2 * jc + 1) * 16)
                    v00 = plsc.load_gather(val_v, [r0, czero])
                    v01 = plsc.load_gather(val_v, [r0, cone])
                    v10 = plsc.load_gather(val_v, [r1, czero])
                    v11 = plsc.load_gather(val_v, [r1, cone])
                    acc0 = acc0 + wj[jc] * (v00 + v10)
                    acc1 = acc1 + wj[jc] * (v01 + v11)
                plsc.store_scatter(
                    out_v, [rows, jnp.full((16,), 3 + 2 * l, jnp.int32)], acc0)
                plsc.store_scatter(
                    out_v, [rows, jnp.full((16,), 4 + 2 * l, jnp.int32)], acc1)

            pltpu.sync_copy(out_v.at[pl.ds(pslot * _i32(_C), _C)],
                            out_hbm.at[pl.ds(tile_base + j * _i32(_C), _C)])

    def _sbody(i, _):
        _it(i)
        return i + _i32(1), None

    lax.scan(_sbody, _i32(0), None, length=_NCHUNK + 1)


@jax.jit
def _embed(xs, ys, zs, dense, hash2d):
    n = xs.shape[0]
    mesh = plsc.VectorSubcoreMesh(core_axis_name="c", subcore_axis_name="s")
    return pl.kernel(
        _body,
        out_type=jax.ShapeDtypeStruct((n, _OD), jnp.float32),
        mesh=mesh,
        compiler_params=pltpu.CompilerParams(
            needs_layout_passes=False, use_tc_tiling_on_sc=False),
        scratch_types=[
            pltpu.VMEM((3 * _PPT,), jnp.float32),
            pltpu.VMEM((2 * _NL, _GROW), jnp.int32),
            pltpu.VMEM((2 * _NL * _GROW, 16), jnp.float32),
            pltpu.VMEM((2 * _C, _OD), jnp.float32),
            pltpu.SemaphoreType.DMA,
        ],
    )(xs, ys, zs, dense, hash2d)


def kernel(xyz, dense, hash_table):
    xyz_t = xyz.astype(jnp.float32).T
    pad = ((0, 0), (0, 14))
    dense_p = jnp.pad(dense.astype(jnp.float32), pad)
    hash_p = jnp.pad(hash_table.astype(jnp.float32).reshape(-1, 2), pad)
    return _embed(xyz_t[0], xyz_t[1], xyz_t[2], dense_p, hash_p)


# speedup vs baseline: 11.4761x; 1.2436x over previous
import functools

import numpy as np
import jax
import jax.numpy as jnp
from jax import lax
from jax.experimental import pallas as pl
from jax.experimental.pallas import tpu as pltpu
from jax.experimental.pallas import tpu_sc as plsc

_NL = 16
_SH = 6
_T = 524309
_RES = [16, 22, 30, 42, 58, 80, 110, 152, 210, 290, 400, 553, 763, 1053, 1453, 2005]
_DBASE = [0, 4096, 14744, 41744, 115832, 310944]
_P1, _P2 = 19349663, 83492791
_P1H, _P1L = _P1 >> 16, _P1 & 0xFFFF
_P2H, _P2L = _P2 >> 16, _P2 & 0xFFFF
_INVT = np.float32(1.0) / np.float32(_T)
_MIN32 = np.int32(-(2 ** 31))
_SCALE = [float(np.float32(1.0) / np.float32(1.0 / (r - 1.0))) for r in _RES]

_C = 16
_NW = 32
_PPT = 4096
_NCHUNK = _PPT // _C
_GROW = 8 * _C
_OD = 3 + 2 * _NL


def _i32(v):
    return np.int32(v)


def _f32(v):
    return np.float32(v)


def _prod16(v, ph, plo):
    m = v * _i32(ph)
    lp = v * _i32(plo)
    a = m << 16
    lo = a + lp
    carry = jnp.where((lo ^ _MIN32) < (a ^ _MIN32), _i32(1), _i32(0))
    hi = (m >> 16) + carry
    return hi, lo


def _mod_t(hi, lo):
    lof = lo.astype(jnp.float32) + jnp.where(lo < 0, _f32(2.0 ** 32), _f32(0.0))
    vf = hi.astype(jnp.float32) * _f32(2.0 ** 32) + lof
    q = (vf * _INVT).astype(jnp.int32)
    r = lo - q * _i32(_T)
    r = r + jnp.where(r < 0, _i32(_T), _i32(0))
    r = r - jnp.where(r >= _T, _i32(_T), _i32(0))
    return r


def _body(x_hbm, y_hbm, z_hbm, dense_hbm, hash_hbm, out_hbm,
          xyz_v, idx_v, val_v, out_v, sem0, sem1):
    wid = lax.axis_index("s") * _i32(2) + lax.axis_index("c")
    tile_base = wid * _i32(_PPT)
    sems = (sem0, sem1)

    iota = lax.iota(jnp.int32, 16)
    czero = jnp.zeros((16,), jnp.int32)
    cone = jnp.full((16,), 1, jnp.int32)

    for d, src in enumerate((x_hbm, y_hbm, z_hbm)):
        pltpu.sync_copy(src.at[pl.ds(tile_base, _PPT)],
                        xyz_v.at[pl.ds(d * _PPT, _PPT)])

    def _norm(j16):
        x = xyz_v[pl.ds(j16, 16)] * _f32(0.5) + _f32(0.5)
        y = xyz_v[pl.ds(j16 + _i32(_PPT), 16)] * _f32(0.5) + _f32(0.5)
        z = xyz_v[pl.ds(j16 + _i32(2 * _PPT), 16)] * _f32(0.5) + _f32(0.5)
        return x, y, z

    def _fire(c, sl):
        x, y, z = _norm(c * _i32(_C))
        for l in range(_NL):
            r = _RES[l]
            fx = x * _f32(_SCALE[l])
            fy = y * _f32(_SCALE[l])
            fz = z * _f32(_SCALE[l])
            ix0 = fx.astype(jnp.int32)
            iy0 = fy.astype(jnp.int32)
            iz0 = fz.astype(jnp.int32)
            ix1 = jnp.minimum((fx + _f32(1.0)).astype(jnp.int32), _i32(r - 1))
            iy1 = jnp.minimum((fy + _f32(1.0)).astype(jnp.int32), _i32(r - 1))
            iz1 = jnp.minimum((fz + _f32(1.0)).astype(jnp.int32), _i32(r - 1))
            row = jnp.int32(sl * _NL + l)
            if l < _SH:
                for k in range(8):
                    xk = ix1 if (k >> 2) & 1 else ix0
                    yk = iy1 if (k >> 1) & 1 else iy0
                    zk = iz1 if k & 1 else iz0
                    idx = xk * _i32(r * r) + yk * _i32(r) + zk + _i32(_DBASE[l])
                    idx_v[row, pl.ds(k * 16, 16)] = idx
            else:
                yp = (_prod16(iy0, _P1H, _P1L), _prod16(iy1, _P1H, _P1L))
                zp = (_prod16(iz0, _P2H, _P2L), _prod16(iz1, _P2H, _P2L))
                xv = (ix0 ^ _i32(1), ix1 ^ _i32(1))
                lbase = _i32((l - _SH) * _T)
                for dy in range(2):
                    for dz in range(2):
                        lo_b = yp[dy][1] ^ zp[dz][1]
                        hi_b = yp[dy][0] ^ zp[dz][0]
                        rb = _mod_t(hi_b, lo_b)
                        b11 = lo_b & _i32(0x7FF)
                        for dx in range(2):
                            rr = rb + ((b11 ^ xv[dx]) - b11)
                            rr = rr + jnp.where(rr < 0, _i32(_T), _i32(0))
                            rr = rr - jnp.where(rr >= _T, _i32(_T), _i32(0))
                            k = (dx << 2) | (dy << 1) | dz
                            idx_v[row, pl.ds(k * 16, 16)] = rr + lbase

        for l in range(_NL):
            tbl = dense_hbm if l < _SH else hash_hbm
            row = sl * _NL + l
            pltpu.async_copy(
                tbl.at[idx_v.at[jnp.int32(row)]],
                val_v.at[pl.ds(row * _GROW, _GROW)],
                sems[sl])

    def _consume(c, sl):
        for l in range(_NL):
            tbl = dense_hbm if l < _SH else hash_hbm
            row = sl * _NL + l
            pltpu.make_async_copy(
                tbl.at[idx_v.at[jnp.int32(row)]],
                val_v.at[pl.ds(row * _GROW, _GROW)],
                sems[sl]).wait()
        x, y, z = _norm(c * _i32(_C))
        rows = _i32(sl * _C) + iota
        plsc.store_scatter(out_v, [rows, czero], x)
        plsc.store_scatter(out_v, [rows, cone], y)
        plsc.store_scatter(out_v, [rows, jnp.full((16,), 2, jnp.int32)], z)
        for l in range(_NL):
            fx = x * _f32(_SCALE[l])
            fy = y * _f32(_SCALE[l])
            wx = fx - fx.astype(jnp.int32).astype(jnp.float32)
            wy = fy - fy.astype(jnp.int32).astype(jnp.float32)
            ux = _f32(1.0) - wx
            uy = _f32(1.0) - wy
            wj = (ux * uy, ux * wy, wx * uy, wx * wy)
            base = _i32((sl * _NL + l) * _GROW) + iota
            acc0 = jnp.zeros((16,), jnp.float32)
            acc1 = jnp.zeros((16,), jnp.float32)
            for jc in range(4):
                r0 = base + _i32((2 * jc) * 16)
                r1 = base + _i32((2 * jc + 1) * 16)
                v00 = plsc.load_gather(val_v, [r0, czero])
                v01 = plsc.load_gather(val_v, [r0, cone])
                v10 = plsc.load_gather(val_v, [r1, czero])
                v11 = plsc.load_gather(val_v, [r1, cone])
                acc0 = acc0 + wj[jc] * (v00 + v10)
                acc1 = acc1 + wj[jc] * (v01 + v11)
            plsc.store_scatter(
                out_v, [rows, jnp.full((16,), 3 + 2 * l, jnp.int32)], acc0)
            plsc.store_scatter(
                out_v, [rows, jnp.full((16,), 4 + 2 * l, jnp.int32)], acc1)

        pltpu.sync_copy(out_v.at[pl.ds(sl * _C, _C)],
                        out_hbm.at[pl.ds(tile_base + c * _i32(_C), _C)])

    _fire(jnp.int32(0), 0)

    def _sbody(c, _):
        _fire(c + _i32(1), 1)
        _consume(c, 0)

        @pl.when(c + _i32(2) < _i32(_NCHUNK))
        def _refill():
            _fire(c + _i32(2), 0)

        _consume(c + _i32(1), 1)
        return c + _i32(2), None

    lax.scan(_sbody, _i32(0), None, length=_NCHUNK // 2)


@jax.jit
def _embed(xs, ys, zs, dense, hash2d):
    n = xs.shape[0]
    mesh = plsc.VectorSubcoreMesh(core_axis_name="c", subcore_axis_name="s")
    return pl.kernel(
        _body,
        out_type=jax.ShapeDtypeStruct((n, _OD), jnp.float32),
        mesh=mesh,
        compiler_params=pltpu.CompilerParams(
            needs_layout_passes=False, use_tc_tiling_on_sc=False),
        scratch_types=[
            pltpu.VMEM((3 * _PPT,), jnp.float32),
            pltpu.VMEM((2 * _NL, _GROW), jnp.int32),
            pltpu.VMEM((2 * _NL * _GROW, 16), jnp.float32),
            pltpu.VMEM((2 * _C, _OD), jnp.float32),
            pltpu.SemaphoreType.DMA,
            pltpu.SemaphoreType.DMA,
        ],
    )(xs, ys, zs, dense, hash2d)


def kernel(xyz, dense, hash_table):
    xyz_t = xyz.astype(jnp.float32).T
    pad = ((0, 0), (0, 14))
    dense_p = jnp.pad(dense.astype(jnp.float32), pad)
    hash_p = jnp.pad(hash_table.astype(jnp.float32).reshape(-1, 2), pad)
    return _embed(xyz_t[0], xyz_t[1], xyz_t[2], dense_p, hash_p)


# speedup vs baseline: 11.5176x; 1.0036x over previous
import functools

import numpy as np
import jax
import jax.numpy as jnp
from jax import lax
from jax.experimental import pallas as pl
from jax.experimental.pallas import tpu as pltpu
from jax.experimental.pallas import tpu_sc as plsc

_NL = 16
_SH = 6
_T = 524309
_RES = [16, 22, 30, 42, 58, 80, 110, 152, 210, 290, 400, 553, 763, 1053, 1453, 2005]
_DBASE = [0, 4096, 14744, 41744, 115832, 310944]
_P1, _P2 = 19349663, 83492791
_P1H, _P1L = _P1 >> 16, _P1 & 0xFFFF
_P2H, _P2L = _P2 >> 16, _P2 & 0xFFFF
_INVT = np.float32(1.0) / np.float32(_T)
_MIN32 = np.int32(-(2 ** 31))
_SCALE = [float(np.float32(1.0) / np.float32(1.0 / (r - 1.0))) for r in _RES]

_C = 16
_NW = 32
_PPT = 4096
_NCHUNK = _PPT // _C
_GROW = 8 * _C
_OD = 3 + 2 * _NL


def _i32(v):
    return np.int32(v)


def _f32(v):
    return np.float32(v)


def _prod16(v, ph, plo):
    m = v * _i32(ph)
    lp = v * _i32(plo)
    a = m << 16
    lo = a + lp
    carry = jnp.where((lo ^ _MIN32) < (a ^ _MIN32), _i32(1), _i32(0))
    hi = (m >> 16) + carry
    return hi, lo


def _mod_t(hi, lo):
    lof = lo.astype(jnp.float32) + jnp.where(lo < 0, _f32(2.0 ** 32), _f32(0.0))
    vf = hi.astype(jnp.float32) * _f32(2.0 ** 32) + lof
    q = (vf * _INVT).astype(jnp.int32)
    r = lo - q * _i32(_T)
    r = r + jnp.where(r < 0, _i32(_T), _i32(0))
    r = r - jnp.where(r >= _T, _i32(_T), _i32(0))
    return r


def _body(x_hbm, y_hbm, z_hbm, dense_hbm, hash_hbm, out_hbm,
          xyz_v, idx_v, val_v, out_v, sem0, sem1, sem2):
    wid = lax.axis_index("s") * _i32(2) + lax.axis_index("c")
    tile_base = wid * _i32(_PPT)
    sems = (sem0, sem1, sem2)

    iota = lax.iota(jnp.int32, 16)
    czero = jnp.zeros((16,), jnp.int32)
    cone = jnp.full((16,), 1, jnp.int32)

    for d, src in enumerate((x_hbm, y_hbm, z_hbm)):
        pltpu.sync_copy(src.at[pl.ds(tile_base, _PPT)],
                        xyz_v.at[pl.ds(d * _PPT, _PPT)])

    def _norm(j16):
        x = xyz_v[pl.ds(j16, 16)] * _f32(0.5) + _f32(0.5)
        y = xyz_v[pl.ds(j16 + _i32(_PPT), 16)] * _f32(0.5) + _f32(0.5)
        z = xyz_v[pl.ds(j16 + _i32(2 * _PPT), 16)] * _f32(0.5) + _f32(0.5)
        return x, y, z

    def _fire(c, sl):
        x, y, z = _norm(c * _i32(_C))
        for l in range(_NL):
            r = _RES[l]
            fx = x * _f32(_SCALE[l])
            fy = y * _f32(_SCALE[l])
            fz = z * _f32(_SCALE[l])
            ix0 = fx.astype(jnp.int32)
            iy0 = fy.astype(jnp.int32)
            iz0 = fz.astype(jnp.int32)
            ix1 = jnp.minimum((fx + _f32(1.0)).astype(jnp.int32), _i32(r - 1))
            iy1 = jnp.minimum((fy + _f32(1.0)).astype(jnp.int32), _i32(r - 1))
            iz1 = jnp.minimum((fz + _f32(1.0)).astype(jnp.int32), _i32(r - 1))
            row = jnp.int32(sl * _NL + l)
            if l < _SH:
                for k in range(8):
                    xk = ix1 if (k >> 2) & 1 else ix0
                    yk = iy1 if (k >> 1) & 1 else iy0
                    zk = iz1 if k & 1 else iz0
                    idx = xk * _i32(r * r) + yk * _i32(r) + zk + _i32(_DBASE[l])
                    idx_v[row, pl.ds(k * 16, 16)] = idx
            else:
                yp = (_prod16(iy0, _P1H, _P1L), _prod16(iy1, _P1H, _P1L))
                zp = (_prod16(iz0, _P2H, _P2L), _prod16(iz1, _P2H, _P2L))
                xv = (ix0 ^ _i32(1), ix1 ^ _i32(1))
                lbase = _i32((l - _SH) * _T)
                for dy in range(2):
                    for dz in range(2):
                        lo_b = yp[dy][1] ^ zp[dz][1]
                        hi_b = yp[dy][0] ^ zp[dz][0]
                        rb = _mod_t(hi_b, lo_b)
                        b11 = lo_b & _i32(0x7FF)
                        for dx in range(2):
                            rr = rb + ((b11 ^ xv[dx]) - b11)
                            rr = rr + jnp.where(rr < 0, _i32(_T), _i32(0))
                            rr = rr - jnp.where(rr >= _T, _i32(_T), _i32(0))
                            k = (dx << 2) | (dy << 1) | dz
                            idx_v[row, pl.ds(k * 16, 16)] = rr + lbase

        for l in range(_NL):
            tbl = dense_hbm if l < _SH else hash_hbm
            row = sl * _NL + l
            pltpu.async_copy(
                tbl.at[idx_v.at[jnp.int32(row)]],
                val_v.at[pl.ds(row * _GROW, _GROW)],
                sems[sl])

    def _consume(c, sl):
        for l in range(_NL):
            tbl = dense_hbm if l < _SH else hash_hbm
            row = sl * _NL + l
            pltpu.make_async_copy(
                tbl.at[idx_v.at[jnp.int32(row)]],
                val_v.at[pl.ds(row * _GROW, _GROW)],
                sems[sl]).wait()
        x, y, z = _norm(c * _i32(_C))
        rows = iota
        plsc.store_scatter(out_v, [rows, czero], x)
        plsc.store_scatter(out_v, [rows, cone], y)
        plsc.store_scatter(out_v, [rows, jnp.full((16,), 2, jnp.int32)], z)
        for l in range(_NL):
            fx = x * _f32(_SCALE[l])
            fy = y * _f32(_SCALE[l])
            wx = fx - fx.astype(jnp.int32).astype(jnp.float32)
            wy = fy - fy.astype(jnp.int32).astype(jnp.float32)
            ux = _f32(1.0) - wx
            uy = _f32(1.0) - wy
            wj = (ux * uy, ux * wy, wx * uy, wx * wy)
            base = _i32((sl * _NL + l) * _GROW) + iota
            acc0 = jnp.zeros((16,), jnp.float32)
            acc1 = jnp.zeros((16,), jnp.float32)
            for jc in range(4):
                r0 = base + _i32((2 * jc) * 16)
                r1 = base + _i32((2 * jc + 1) * 16)
                v00 = plsc.load_gather(val_v, [r0, czero])
                v01 = plsc.load_gather(val_v, [r0, cone])
                v10 = plsc.load_gather(val_v, [r1, czero])
                v11 = plsc.load_gather(val_v, [r1, cone])
                acc0 = acc0 + wj[jc] * (v00 + v10)
                acc1 = acc1 + wj[jc] * (v01 + v11)
            plsc.store_scatter(
                out_v, [rows, jnp.full((16,), 3 + 2 * l, jnp.int32)], acc0)
            plsc.store_scatter(
                out_v, [rows, jnp.full((16,), 4 + 2 * l, jnp.int32)], acc1)

        pltpu.sync_copy(out_v,
                        out_hbm.at[pl.ds(tile_base + c * _i32(_C), _C)])

    _fire(jnp.int32(0), 0)
    _fire(jnp.int32(1), 1)

    def _sbody(c, _):
        _fire(c + _i32(2), 2)
        _consume(c, 0)
        _fire(c + _i32(3), 0)
        _consume(c + _i32(1), 1)

        @pl.when(c + _i32(4) < _i32(_NCHUNK))
        def _refill():
            _fire(c + _i32(4), 1)

        _consume(c + _i32(2), 2)
        return c + _i32(3), None

    lax.scan(_sbody, _i32(0), None, length=_NCHUNK // 3)
    _consume(jnp.int32(_NCHUNK - 1), 0)


@jax.jit
def _embed(xs, ys, zs, dense, hash2d):
    n = xs.shape[0]
    mesh = plsc.VectorSubcoreMesh(core_axis_name="c", subcore_axis_name="s")
    return pl.kernel(
        _body,
        out_type=jax.ShapeDtypeStruct((n, _OD), jnp.float32),
        mesh=mesh,
        compiler_params=pltpu.CompilerParams(
            needs_layout_passes=False, use_tc_tiling_on_sc=False),
        scratch_types=[
            pltpu.VMEM((3 * _PPT,), jnp.float32),
            pltpu.VMEM((3 * _NL, _GROW), jnp.int32),
            pltpu.VMEM((3 * _NL * _GROW, 16), jnp.float32),
            pltpu.VMEM((_C, _OD), jnp.float32),
            pltpu.SemaphoreType.DMA,
            pltpu.SemaphoreType.DMA,
            pltpu.SemaphoreType.DMA,
        ],
    )(xs, ys, zs, dense, hash2d)


def kernel(xyz, dense, hash_table):
    xyz_t = xyz.astype(jnp.float32).T
    pad = ((0, 0), (0, 14))
    dense_p = jnp.pad(dense.astype(jnp.float32), pad)
    hash_p = jnp.pad(hash_table.astype(jnp.float32).reshape(-1, 2), pad)
    return _embed(xyz_t[0], xyz_t[1], xyz_t[2], dense_p, hash_p)
